# baseline (speedup 1.0000x reference)
import functools

import jax
import jax.numpy as jnp
from jax import lax
from jax.experimental import pallas as pl
from jax.experimental.pallas import tpu as pltpu
from jax.experimental.pallas import tpu_sc as plsc

B = 8
N = 2048
D = 128
K = 512
JCHUNK = 512
RCHUNK = 16
ROWS_PER_TILE = (B * K) // 32


def _topk_body(s_row_ref, s_col_ref, idx_ref, val_ref):
    s_row = s_row_ref[0]
    s_col_full = s_col_ref[0]

    rank = jnp.zeros((1, N), jnp.float32)
    for c in range(N // JCHUNK):
        s_j = s_col_ref[0, pl.ds(c * JCHUNK, JCHUNK), :]
        j_ids = c * JCHUNK + lax.broadcasted_iota(jnp.int32, (JCHUNK, N), 0)
        i_ids = lax.broadcasted_iota(jnp.int32, (JCHUNK, N), 1)
        gt = s_j > s_row
        tie = (s_j == s_row) & (j_ids < i_ids)
        rank = rank + jnp.sum((gt | tie).astype(jnp.float32), axis=0,
                              keepdims=True)

    p_ids = lax.broadcasted_iota(jnp.int32, (K, N), 0).astype(jnp.float32)
    E = (jnp.broadcast_to(rank, (K, N)) == p_ids).astype(jnp.float32)

    i_row = lax.broadcasted_iota(jnp.int32, (K, N), 1).astype(jnp.float32)
    idx_f = jnp.sum(E * i_row, axis=1, keepdims=True)
    vals = jnp.sum(E * jnp.broadcast_to(s_row, (K, N)), axis=1, keepdims=True)
    idx_ref[0] = idx_f.astype(jnp.int32)
    val_ref[0] = vals


def _topk_call(scores_row, scores_col, interpret=False):
    return pl.pallas_call(
        _topk_body,
        grid=(B,),
        in_specs=[
            pl.BlockSpec((1, 1, N), lambda b: (b, 0, 0)),
            pl.BlockSpec((1, N, 1), lambda b: (b, 0, 0)),
        ],
        out_specs=[
            pl.BlockSpec((1, K, 1), lambda b: (b, 0, 0)),
            pl.BlockSpec((1, K, 1), lambda b: (b, 0, 0)),
        ],
        out_shape=[
            jax.ShapeDtypeStruct((B, K, 1), jnp.int32),
            jax.ShapeDtypeStruct((B, K, 1), jnp.float32),
        ],
        interpret=interpret,
    )(scores_row, scores_col)


def _sc_gather_body(g_hbm, h_hbm, rowidx_hbm, colidx_hbm,
                    outg_hbm, outh_hbm,
                    colv, rowv, grows, hrows, outg_v,
                    gsem, hsem):
    nc = 2
    wid = lax.axis_index("s") * nc + lax.axis_index("c")
    b = wid // (K // ROWS_PER_TILE)
    out_base = pl.multiple_of(wid * ROWS_PER_TILE, ROWS_PER_TILE)

    pltpu.sync_copy(colidx_hbm.at[pl.ds(pl.multiple_of(b * K, K), K)], colv)
    pltpu.sync_copy(rowidx_hbm.at[pl.ds(out_base, ROWS_PER_TILE)], rowv)

    for c in range(ROWS_PER_TILE // RCHUNK):
        ridx = rowv.at[pl.ds(c * RCHUNK, RCHUNK)]
        cp_g = pltpu.async_copy(g_hbm.at[ridx], grows, gsem)
        cp_h = pltpu.async_copy(h_hbm.at[ridx], hrows, hsem)
        cp_g.wait()

        def row_body(r, carry):
            r_splat = jnp.full((16,), 0, jnp.int32) + r
            for cc in range(K // 16):
                cidx = colv[pl.ds(cc * 16, 16)]
                vals = plsc.load_gather(
                    grows,
                    [r_splat, lax.shift_right_logical(cidx, 7),
                     lax.bitwise_and(cidx, 127)])
                outg_v[r, cc >> 3, pl.ds((cc & 7) * 16, 16)] = vals
            return carry

        lax.fori_loop(0, RCHUNK, row_body, 0)

        out_off = pl.multiple_of(out_base + c * RCHUNK, RCHUNK)
        pltpu.sync_copy(outg_v, outg_hbm.at[pl.ds(out_off, RCHUNK)])
        cp_h.wait()
        pltpu.sync_copy(hrows, outh_hbm.at[pl.ds(out_off, RCHUNK)])


def _sc_gather(g3, h_flat, rowidx, colidx):
    mesh = plsc.VectorSubcoreMesh(core_axis_name="c", subcore_axis_name="s")
    fn = pl.kernel(
        _sc_gather_body,
        out_type=[
            jax.ShapeDtypeStruct((B * K, K // 128, 128), jnp.float32),
            jax.ShapeDtypeStruct((B * K, D), jnp.float32),
        ],
        mesh=mesh,
        scratch_types=[
            pltpu.VMEM((K,), jnp.int32),
            pltpu.VMEM((ROWS_PER_TILE,), jnp.int32),
            pltpu.VMEM((RCHUNK, N // 128, 128), jnp.float32),
            pltpu.VMEM((RCHUNK, D), jnp.float32),
            pltpu.VMEM((RCHUNK, K // 128, 128), jnp.float32),
            pltpu.SemaphoreType.DMA,
            pltpu.SemaphoreType.DMA,
        ],
        compiler_params=pltpu.CompilerParams(use_tc_tiling_on_sc=True,
                                             needs_layout_passes=False),
    )
    return fn(g3, h_flat, rowidx, colidx)


@jax.jit
def kernel(h, g, scores):
    s2 = scores[:, 0, :, 0]
    idx3, val3 = _topk_call(s2.reshape(B, 1, N), s2.reshape(B, N, 1))
    idx = idx3[:, :, 0]

    row_flat = (idx + jnp.arange(B, dtype=jnp.int32)[:, None] * N).reshape(-1)
    col_flat = idx.reshape(-1)

    g3 = g.reshape(B * N, N // 128, 128)
    h_flat = h.reshape(B * N, D)
    outg, outh = _sc_gather(g3, h_flat, row_flat, col_flat)

    hs = outh.reshape(B, 1, K, D)
    gs = outg.reshape(B, 1, K, K)
    ss = val3.reshape(B, 1, K)
    return (hs, gs, ss)


# speedup vs baseline: 1.5765x; 1.5765x over previous
import functools

import jax
import jax.numpy as jnp
from jax import lax
from jax.experimental import pallas as pl
from jax.experimental.pallas import tpu as pltpu
from jax.experimental.pallas import tpu_sc as plsc

B = 8
N = 2048
D = 128
K = 512
JCHUNK = 512
RCHUNK = 16
ROWS_PER_TILE = (B * K) // 32


def _topk_body(s_row_ref, s_col_ref, idx_ref, val_ref):
    s_row = s_row_ref[0]
    s_col_full = s_col_ref[0]

    rank = jnp.zeros((1, N), jnp.float32)
    for c in range(N // JCHUNK):
        s_j = s_col_ref[0, pl.ds(c * JCHUNK, JCHUNK), :]
        j_ids = c * JCHUNK + lax.broadcasted_iota(jnp.int32, (JCHUNK, N), 0)
        i_ids = lax.broadcasted_iota(jnp.int32, (JCHUNK, N), 1)
        gt = s_j > s_row
        tie = (s_j == s_row) & (j_ids < i_ids)
        rank = rank + jnp.sum((gt | tie).astype(jnp.float32), axis=0,
                              keepdims=True)

    p_ids = lax.broadcasted_iota(jnp.int32, (K, N), 0).astype(jnp.float32)
    E = (jnp.broadcast_to(rank, (K, N)) == p_ids).astype(jnp.float32)

    i_row = lax.broadcasted_iota(jnp.int32, (K, N), 1).astype(jnp.float32)
    idx_f = jnp.sum(E * i_row, axis=1, keepdims=True)
    vals = jnp.sum(E * jnp.broadcast_to(s_row, (K, N)), axis=1, keepdims=True)
    idx_ref[0] = idx_f.astype(jnp.int32)
    val_ref[0] = vals


def _topk_call(scores_row, scores_col, interpret=False):
    return pl.pallas_call(
        _topk_body,
        grid=(B,),
        in_specs=[
            pl.BlockSpec((1, 1, N), lambda b: (b, 0, 0)),
            pl.BlockSpec((1, N, 1), lambda b: (b, 0, 0)),
        ],
        out_specs=[
            pl.BlockSpec((1, K, 1), lambda b: (b, 0, 0)),
            pl.BlockSpec((1, K, 1), lambda b: (b, 0, 0)),
        ],
        out_shape=[
            jax.ShapeDtypeStruct((B, K, 1), jnp.int32),
            jax.ShapeDtypeStruct((B, K, 1), jnp.float32),
        ],
        interpret=interpret,
    )(scores_row, scores_col)


def _sc_gather_body(g_hbm, h_hbm, rowidx_hbm, colidx_hbm,
                    outg_hbm, outh_hbm,
                    colv, rowv, grows, hrows, outg_v,
                    gsem, hsem):
    nc = 2
    wid = lax.axis_index("s") * nc + lax.axis_index("c")
    b = wid // (K // ROWS_PER_TILE)
    out_base = pl.multiple_of(wid * ROWS_PER_TILE, ROWS_PER_TILE)

    pltpu.sync_copy(colidx_hbm.at[pl.ds(pl.multiple_of(b * K, K), K)], colv)
    pltpu.sync_copy(rowidx_hbm.at[pl.ds(out_base, ROWS_PER_TILE)], rowv)

    for c in range(ROWS_PER_TILE // RCHUNK):
        ridx = rowv.at[pl.ds(c * RCHUNK, RCHUNK)]
        cp_g = pltpu.async_copy(g_hbm.at[ridx], grows, gsem)
        cp_h = pltpu.async_copy(h_hbm.at[ridx], hrows, hsem)
        cp_g.wait()

        def row_body(r, carry):
            r_splat = jnp.full((16,), 0, jnp.int32) + r
            for cc in range(K // 16):
                cidx = colv[pl.ds(cc * 16, 16)]
                vals = plsc.load_gather(grows, [r_splat, cidx])
                outg_v[r, pl.ds(cc * 16, 16)] = vals
            return carry

        lax.fori_loop(0, RCHUNK, row_body, 0)

        out_off = pl.multiple_of(out_base + c * RCHUNK, RCHUNK)
        pltpu.sync_copy(outg_v, outg_hbm.at[pl.ds(out_off, RCHUNK)])
        cp_h.wait()
        pltpu.sync_copy(hrows, outh_hbm.at[pl.ds(out_off, RCHUNK)])


def _sc_gather(g3, h_flat, rowidx, colidx):
    mesh = plsc.VectorSubcoreMesh(core_axis_name="c", subcore_axis_name="s")
    fn = pl.kernel(
        _sc_gather_body,
        out_type=[
            jax.ShapeDtypeStruct((B * K, K), jnp.float32),
            jax.ShapeDtypeStruct((B * K, D), jnp.float32),
        ],
        mesh=mesh,
        scratch_types=[
            pltpu.VMEM((K,), jnp.int32),
            pltpu.VMEM((ROWS_PER_TILE,), jnp.int32),
            pltpu.VMEM((RCHUNK, N), jnp.float32),
            pltpu.VMEM((RCHUNK, D), jnp.float32),
            pltpu.VMEM((RCHUNK, K), jnp.float32),
            pltpu.SemaphoreType.DMA,
            pltpu.SemaphoreType.DMA,
        ],
        compiler_params=pltpu.CompilerParams(use_tc_tiling_on_sc=True,
                                             needs_layout_passes=False),
    )
    return fn(g3, h_flat, rowidx, colidx)


@jax.jit
def kernel(h, g, scores):
    s2 = scores[:, 0, :, 0]
    idx3, val3 = _topk_call(s2.reshape(B, 1, N), s2.reshape(B, N, 1))
    idx = idx3[:, :, 0]

    row_flat = (idx + jnp.arange(B, dtype=jnp.int32)[:, None] * N).reshape(-1)
    col_flat = idx.reshape(-1)

    g_flat = g.reshape(B * N, N)
    h_flat = h.reshape(B * N, D)
    outg, outh = _sc_gather(g_flat, h_flat, row_flat, col_flat)

    hs = outh.reshape(B, 1, K, D)
    gs = outg.reshape(B, 1, K, K)
    ss = val3.reshape(B, 1, K)
    return (hs, gs, ss)


# speedup vs baseline: 1.8265x; 1.1586x over previous
import functools

import jax
import jax.numpy as jnp
from jax import lax
from jax.experimental import pallas as pl
from jax.experimental.pallas import tpu as pltpu
from jax.experimental.pallas import tpu_sc as plsc

B = 8
N = 2048
D = 128
K = 512
JCHUNK = 512
RCHUNK = 16
ROWS_PER_TILE = (B * K) // 32


def _topk_body(s_row_ref, s_col_ref, idx_ref, val_ref):
    s_row = s_row_ref[0]
    s_col_full = s_col_ref[0]

    rank = jnp.zeros((1, N), jnp.float32)
    for c in range(N // JCHUNK):
        s_j = s_col_ref[0, pl.ds(c * JCHUNK, JCHUNK), :]
        j_ids = c * JCHUNK + lax.broadcasted_iota(jnp.int32, (JCHUNK, N), 0)
        i_ids = lax.broadcasted_iota(jnp.int32, (JCHUNK, N), 1)
        gt = s_j > s_row
        tie = (s_j == s_row) & (j_ids < i_ids)
        rank = rank + jnp.sum((gt | tie).astype(jnp.float32), axis=0,
                              keepdims=True)

    p_ids = lax.broadcasted_iota(jnp.int32, (K, N), 0).astype(jnp.float32)
    E = (jnp.broadcast_to(rank, (K, N)) == p_ids).astype(jnp.float32)

    i_row = lax.broadcasted_iota(jnp.int32, (K, N), 1).astype(jnp.float32)
    idx_f = jnp.sum(E * i_row, axis=1, keepdims=True)
    vals = jnp.sum(E * jnp.broadcast_to(s_row, (K, N)), axis=1, keepdims=True)
    idx_ref[0] = idx_f.astype(jnp.int32)
    val_ref[0] = vals


def _topk_call(scores_row, scores_col, interpret=False):
    return pl.pallas_call(
        _topk_body,
        grid=(B,),
        in_specs=[
            pl.BlockSpec((1, 1, N), lambda b: (b, 0, 0)),
            pl.BlockSpec((1, N, 1), lambda b: (b, 0, 0)),
        ],
        out_specs=[
            pl.BlockSpec((1, K, 1), lambda b: (b, 0, 0)),
            pl.BlockSpec((1, K, 1), lambda b: (b, 0, 0)),
        ],
        out_shape=[
            jax.ShapeDtypeStruct((B, K, 1), jnp.int32),
            jax.ShapeDtypeStruct((B, K, 1), jnp.float32),
        ],
        interpret=interpret,
    )(scores_row, scores_col)


def _sc_gather_body(g_hbm, h_hbm, rowidx_hbm, colidx_hbm,
                    outg_hbm, outh_hbm,
                    colv, rowv, grows, hrows, outg_v,
                    gsems, hsems, ogsems, ohsems):
    nc = 2
    wid = lax.axis_index("s") * nc + lax.axis_index("c")
    b = wid // (K // ROWS_PER_TILE)
    out_base = pl.multiple_of(wid * ROWS_PER_TILE, ROWS_PER_TILE)

    pltpu.sync_copy(colidx_hbm.at[pl.ds(pl.multiple_of(b * K, K), K)], colv)
    pltpu.sync_copy(rowidx_hbm.at[pl.ds(out_base, ROWS_PER_TILE)], rowv)

    n_chunks = ROWS_PER_TILE // RCHUNK

    def start_gather(c):
        ridx = rowv.at[pl.ds(c * RCHUNK, RCHUNK)]
        s = c % 2
        return (pltpu.async_copy(g_hbm.at[ridx], grows.at[s], gsems.at[s]),
                pltpu.async_copy(h_hbm.at[ridx], hrows.at[s], hsems.at[s]))

    in_flight = start_gather(0)
    out_cps = [None, None]
    for c in range(n_chunks):
        s = c % 2
        cp_g, cp_h = in_flight
        if c + 1 < n_chunks:
            in_flight = start_gather(c + 1)
        cp_g.wait()
        if out_cps[s] is not None:
            out_cps[s][0].wait()
            out_cps[s][1].wait()

        def row_body(r, carry):
            r_splat = jnp.full((16,), 0, jnp.int32) + r
            for cc in range(K // 16):
                cidx = colv[pl.ds(cc * 16, 16)]
                vals = plsc.load_gather(grows.at[s], [r_splat, cidx])
                outg_v[s, r, pl.ds(cc * 16, 16)] = vals
            return carry

        lax.fori_loop(0, RCHUNK, row_body, 0)
        cp_h.wait()

        out_off = pl.multiple_of(out_base + c * RCHUNK, RCHUNK)
        out_cps[s] = (
            pltpu.async_copy(outg_v.at[s], outg_hbm.at[pl.ds(out_off, RCHUNK)],
                             ogsems.at[s]),
            pltpu.async_copy(hrows.at[s], outh_hbm.at[pl.ds(out_off, RCHUNK)],
                             ohsems.at[s]),
        )
    for cps in out_cps:
        if cps is not None:
            cps[0].wait()
            cps[1].wait()


def _sc_gather(g3, h_flat, rowidx, colidx):
    mesh = plsc.VectorSubcoreMesh(core_axis_name="c", subcore_axis_name="s")
    fn = pl.kernel(
        _sc_gather_body,
        out_type=[
            jax.ShapeDtypeStruct((B * K, K), jnp.float32),
            jax.ShapeDtypeStruct((B * K, D), jnp.float32),
        ],
        mesh=mesh,
        scratch_types=[
            pltpu.VMEM((K,), jnp.int32),
            pltpu.VMEM((ROWS_PER_TILE,), jnp.int32),
            pltpu.VMEM((2, RCHUNK, N), jnp.float32),
            pltpu.VMEM((2, RCHUNK, D), jnp.float32),
            pltpu.VMEM((2, RCHUNK, K), jnp.float32),
            pltpu.SemaphoreType.DMA((2,)),
            pltpu.SemaphoreType.DMA((2,)),
            pltpu.SemaphoreType.DMA((2,)),
            pltpu.SemaphoreType.DMA((2,)),
        ],
        compiler_params=pltpu.CompilerParams(use_tc_tiling_on_sc=True,
                                             needs_layout_passes=False),
    )
    return fn(g3, h_flat, rowidx, colidx)


@jax.jit
def kernel(h, g, scores):
    s2 = scores[:, 0, :, 0]
    idx3, val3 = _topk_call(s2.reshape(B, 1, N), s2.reshape(B, N, 1))
    idx = idx3[:, :, 0]

    row_flat = (idx + jnp.arange(B, dtype=jnp.int32)[:, None] * N).reshape(-1)
    col_flat = idx.reshape(-1)

    g_flat = g.reshape(B * N, N)
    h_flat = h.reshape(B * N, D)
    outg, outh = _sc_gather(g_flat, h_flat, row_flat, col_flat)

    hs = outh.reshape(B, 1, K, D)
    gs = outg.reshape(B, 1, K, K)
    ss = val3.reshape(B, 1, K)
    return (hs, gs, ss)
